# baseline (speedup 1.0000x reference)
import functools

import jax
import jax.numpy as jnp
from jax import lax
from jax.experimental import pallas as pl
from jax.experimental.pallas import tpu as pltpu
from jax.experimental.pallas import tpu_sc as plsc

_N = 10000
_E = 320000
_D = 128
_T = 10

_NC = 2
_NS = 16
_NW = _NC * _NS
_CH = 128
_CPP = 32
_NCH0 = 32
_NCH1 = 128
_TOTCH = _NS * (_NCH0 + _NCH1)
_EPAD = _TOTCH * _CH
_NP = 10240
_RPT = _NP // _NS
_ZCH = _RPT // _CH

_BN = 1024
_GRID = _NP // _BN


_NB = 2
_K = 1


def _sc_agg_body(h_hbm, src_hbm, dst_hbm, zeros_hbm, out_hbm,
                 src_v, dst_v, rows_v, gsem, ssem, agg_sh):
    c = lax.axis_index("c")
    s = lax.axis_index("s")
    tid = c * _NS + s

    pltpu.sync_copy(zeros_hbm, rows_v.at[0])
    for k in range(_ZCH):
        pltpu.sync_copy(rows_v.at[0], agg_sh.at[pl.ds(s * _RPT + k * _CH, _CH)])
    plsc.subcore_barrier()

    def _gather_start(j, b):
        pltpu.async_copy(h_hbm.at[src_v.at[j]], rows_v.at[b], gsem.at[b])

    def _gather_wait(j, b):
        pltpu.make_async_copy(
            h_hbm.at[src_v.at[j]], rows_v.at[b], gsem.at[b]).wait()

    def _scatter_start(j, b):
        pltpu.async_copy(rows_v.at[b], agg_sh.at[dst_v.at[j]], ssem.at[b],
                         add=True)

    def _scatter_wait(j, b):
        pltpu.make_async_copy(
            rows_v.at[b], agg_sh.at[dst_v.at[j]], ssem.at[b]).wait()

    _STEADY = ((_CPP - _NB) // _NB) * _NB

    def _run_phase(base):
        pltpu.sync_copy(src_hbm.at[pl.ds(base, _CPP)], src_v)
        pltpu.sync_copy(dst_hbm.at[pl.ds(base, _CPP)], dst_v)

        for t in range(_NB):
            _gather_start(t, t % _NB)
            i = t - _K
            if i >= 0:
                _gather_wait(i, i % _NB)
                _scatter_start(i, i % _NB)

        def outer(gi, carry):
            for v in range(_NB):
                t = _NB + gi * _NB + v
                _scatter_wait(t - _NB, v)
                _gather_start(t, v)
                i = t - _K
                bi = (v - _K) % _NB
                _gather_wait(i, bi)
                _scatter_start(i, bi)
            return carry

        lax.fori_loop(0, _STEADY // _NB, outer, 0)

        for t in range(_NB + _STEADY, _CPP + _NB):
            if 0 <= t - _NB < _CPP:
                _scatter_wait(t - _NB, t % _NB)
            if t < _CPP:
                _gather_start(t, t % _NB)
            i = t - _K
            if 0 <= i < _CPP:
                _gather_wait(i, i % _NB)
                _scatter_start(i, i % _NB)

    @pl.when(c == 0)
    def _():
        for p in range(_NCH0 // _CPP):
            _run_phase(s * _NCH0 + p * _CPP)

    @pl.when(c == 1)
    def _():
        for p in range(_NCH1 // _CPP):
            _run_phase(_NS * _NCH0 + s * _NCH1 + p * _CPP)

    plsc.subcore_barrier()

    pltpu.sync_copy(agg_sh.at[pl.ds(s * _RPT, _RPT)],
                    out_hbm.at[c].at[pl.ds(s * _RPT, _RPT)])


def _sc_aggregate(h_p, src2d, dst2d, zeros_blk):
    return pl.kernel(
        _sc_agg_body,
        out_type=jax.ShapeDtypeStruct((_NC, _NP, _D), jnp.float32),
        mesh=plsc.VectorSubcoreMesh(core_axis_name="c", subcore_axis_name="s"),
        scratch_types=[
            pltpu.VMEM((_CPP, _CH), jnp.int32),
            pltpu.VMEM((_CPP, _CH), jnp.int32),
            pltpu.VMEM((_NB, _CH, _D), jnp.float32),
            pltpu.SemaphoreType.DMA((_NB,)),
            pltpu.SemaphoreType.DMA((_NB,)),
            pltpu.VMEM_SHARED((_NP, _D), jnp.float32),
        ],
    )(h_p, src2d, dst2d, zeros_blk)


def _mlp2_body(x_ref, w1_ref, b1_ref, w2_ref, b2_ref, o_ref):
    h = jnp.dot(x_ref[...], w1_ref[...], preferred_element_type=jnp.float32)
    h = jnp.maximum(h + b1_ref[...], 0.0)
    h = jnp.dot(h, w2_ref[...], preferred_element_type=jnp.float32)
    o_ref[...] = jnp.maximum(h + b2_ref[...], 0.0)


def _gin_body(agg_ref, h_ref, hc_ref, w_ref, b_ref, hn_ref, hcn_ref):
    a = agg_ref[0] + agg_ref[1] + h_ref[...]
    hn = jnp.dot(a, w_ref[...], preferred_element_type=jnp.float32)
    hn = jnp.maximum(hn + b_ref[...], 0.0)
    hn_ref[...] = hn
    hcn_ref[...] = hc_ref[...] + hn


def _gin_final_body(agg_ref, h_ref, hc_ref, w_ref, b_ref, wo_ref, bo_ref,
                    out_ref):
    a = agg_ref[0] + agg_ref[1] + h_ref[...]
    hn = jnp.dot(a, w_ref[...], preferred_element_type=jnp.float32)
    hn = jnp.maximum(hn + b_ref[...], 0.0)
    hc = hc_ref[...] + hn
    out_ref[...] = jnp.dot(hc, wo_ref[...], preferred_element_type=jnp.float32) + bo_ref[...]


_row_spec = pl.BlockSpec((_BN, _D), lambda i: (i, 0))
_w_spec = pl.BlockSpec((_D, _D), lambda i: (0, 0))
_b_spec = pl.BlockSpec((1, _D), lambda i: (0, 0))
_agg_spec = pl.BlockSpec((_NC, _BN, _D), lambda i: (0, i, 0))
_row_shape = jax.ShapeDtypeStruct((_NP, _D), jnp.float32)


def _mlp2(x_p, W1, b1, W2, b2):
    return pl.pallas_call(
        _mlp2_body,
        grid=(_GRID,),
        in_specs=[_row_spec, _w_spec, _b_spec, _w_spec, _b_spec],
        out_specs=_row_spec,
        out_shape=_row_shape,
    )(x_p, W1, b1, W2, b2)


def _gin_layer(agg, h_p, hc_p, Wl, bl):
    return pl.pallas_call(
        _gin_body,
        grid=(_GRID,),
        in_specs=[_agg_spec, _row_spec, _row_spec, _w_spec, _b_spec],
        out_specs=[_row_spec, _row_spec],
        out_shape=[_row_shape, _row_shape],
    )(agg, h_p, hc_p, Wl, bl)


def _gin_final(agg, h_p, hc_p, Wl, bl, Wo_p, bo_p):
    return pl.pallas_call(
        _gin_final_body,
        grid=(_GRID,),
        in_specs=[_agg_spec, _row_spec, _row_spec, _w_spec, _b_spec,
                  _w_spec, _b_spec],
        out_specs=_row_spec,
        out_shape=_row_shape,
    )(agg, h_p, hc_p, Wl, bl, Wo_p, bo_p)


def kernel(x, edge_index, W1, b1, W2, b2, Wl0, bl0, Wl1, bl1, Wl2, bl2,
           Wout, bout):
    x_p = jnp.pad(x, ((0, _NP - _N), (0, 0)))
    pad_e = _EPAD - _E
    src2d = jnp.concatenate(
        [edge_index[0], jnp.full((pad_e,), _N, jnp.int32)]).reshape(
            _TOTCH, _CH)
    dst2d = jnp.concatenate(
        [edge_index[1], jnp.full((pad_e,), _N, jnp.int32)]).reshape(
            _TOTCH, _CH)
    zeros_blk = jnp.zeros((_CH, _D), jnp.float32)
    b1r = b1.reshape(1, _D)
    b2r = b2.reshape(1, _D)
    blr = [bl0.reshape(1, _D), bl1.reshape(1, _D), bl2.reshape(1, _D)]
    Wls = [Wl0, Wl1, Wl2]
    Wo_p = jnp.pad(Wout, ((0, 0), (0, _D - _T)))
    bo_p = jnp.pad(bout, (0, _D - _T)).reshape(1, _D)

    h = _mlp2(x_p, W1, b1r, W2, b2r)
    hc = h
    for layer in range(2):
        agg = _sc_aggregate(h, src2d, dst2d, zeros_blk)
        h, hc = _gin_layer(agg, h, hc, Wls[layer], blr[layer])
    agg = _sc_aggregate(h, src2d, dst2d, zeros_blk)
    out_p = _gin_final(agg, h, hc, Wls[2], blr[2], Wo_p, bo_p)
    return out_p[:_N, :_T]


# speedup vs baseline: 1.1169x; 1.1169x over previous
import functools

import jax
import jax.numpy as jnp
from jax import lax
from jax.experimental import pallas as pl
from jax.experimental.pallas import tpu as pltpu
from jax.experimental.pallas import tpu_sc as plsc

_N = 10000
_E = 320000
_D = 128
_T = 10

_NC = 2
_NS = 16
_NW = _NC * _NS
_CH = 128
_CPP = 32
_NCH0 = 128
_NCH1 = 32
_TOTCH = _NS * (_NCH0 + _NCH1)
_EPAD = _TOTCH * _CH
_NP = 10240
_RPT = _NP // _NS
_ZCH = _RPT // _CH

_BN = 1024
_GRID = _NP // _BN


_NB = 2
_K = 1


def _sc_agg_body(h_hbm, src_hbm, dst_hbm, zeros_hbm, out_hbm,
                 src_v, dst_v, rows_v, gsem, ssem, agg_sh):
    c = lax.axis_index("c")
    s = lax.axis_index("s")
    tid = c * _NS + s

    pltpu.sync_copy(zeros_hbm, rows_v.at[0])
    for k in range(_ZCH):
        pltpu.sync_copy(rows_v.at[0], agg_sh.at[pl.ds(s * _RPT + k * _CH, _CH)])
    plsc.subcore_barrier()

    def _gather_start(j, b):
        pltpu.async_copy(h_hbm.at[src_v.at[j]], rows_v.at[b], gsem.at[b])

    def _gather_wait(j, b):
        pltpu.make_async_copy(
            h_hbm.at[src_v.at[j]], rows_v.at[b], gsem.at[b]).wait()

    def _scatter_start(j, b):
        pltpu.async_copy(rows_v.at[b], agg_sh.at[dst_v.at[j]], ssem.at[b],
                         add=True)

    def _scatter_wait(j, b):
        pltpu.make_async_copy(
            rows_v.at[b], agg_sh.at[dst_v.at[j]], ssem.at[b]).wait()

    _STEADY = ((_CPP - _NB) // _NB) * _NB

    def _run_phase(base):
        pltpu.sync_copy(src_hbm.at[pl.ds(base, _CPP)], src_v)
        pltpu.sync_copy(dst_hbm.at[pl.ds(base, _CPP)], dst_v)

        for t in range(_NB):
            _gather_start(t, t % _NB)
            i = t - _K
            if i >= 0:
                _gather_wait(i, i % _NB)
                _scatter_start(i, i % _NB)

        def outer(gi, carry):
            for v in range(_NB):
                t = _NB + gi * _NB + v
                _scatter_wait(t - _NB, v)
                _gather_start(t, v)
                i = t - _K
                bi = (v - _K) % _NB
                _gather_wait(i, bi)
                _scatter_start(i, bi)
            return carry

        lax.fori_loop(0, _STEADY // _NB, outer, 0)

        for t in range(_NB + _STEADY, _CPP + _NB):
            if 0 <= t - _NB < _CPP:
                _scatter_wait(t - _NB, t % _NB)
            if t < _CPP:
                _gather_start(t, t % _NB)
            i = t - _K
            if 0 <= i < _CPP:
                _gather_wait(i, i % _NB)
                _scatter_start(i, i % _NB)

    @pl.when(c == 0)
    def _():
        for p in range(_NCH0 // _CPP):
            _run_phase(s * _NCH0 + p * _CPP)

    @pl.when(c == 1)
    def _():
        for p in range(_NCH1 // _CPP):
            _run_phase(_NS * _NCH0 + s * _NCH1 + p * _CPP)

    plsc.subcore_barrier()

    pltpu.sync_copy(agg_sh.at[pl.ds(s * _RPT, _RPT)],
                    out_hbm.at[c].at[pl.ds(s * _RPT, _RPT)])


def _sc_aggregate(h_p, src2d, dst2d, zeros_blk):
    return pl.kernel(
        _sc_agg_body,
        out_type=jax.ShapeDtypeStruct((_NC, _NP, _D), jnp.float32),
        mesh=plsc.VectorSubcoreMesh(core_axis_name="c", subcore_axis_name="s"),
        scratch_types=[
            pltpu.VMEM((_CPP, _CH), jnp.int32),
            pltpu.VMEM((_CPP, _CH), jnp.int32),
            pltpu.VMEM((_NB, _CH, _D), jnp.float32),
            pltpu.SemaphoreType.DMA((_NB,)),
            pltpu.SemaphoreType.DMA((_NB,)),
            pltpu.VMEM_SHARED((_NP, _D), jnp.float32),
        ],
    )(h_p, src2d, dst2d, zeros_blk)


def _mlp2_body(x_ref, w1_ref, b1_ref, w2_ref, b2_ref, o_ref):
    h = jnp.dot(x_ref[...], w1_ref[...], preferred_element_type=jnp.float32)
    h = jnp.maximum(h + b1_ref[...], 0.0)
    h = jnp.dot(h, w2_ref[...], preferred_element_type=jnp.float32)
    o_ref[...] = jnp.maximum(h + b2_ref[...], 0.0)


def _gin_body(agg_ref, h_ref, hc_ref, w_ref, b_ref, hn_ref, hcn_ref):
    a = agg_ref[0] + agg_ref[1] + h_ref[...]
    hn = jnp.dot(a, w_ref[...], preferred_element_type=jnp.float32)
    hn = jnp.maximum(hn + b_ref[...], 0.0)
    hn_ref[...] = hn
    hcn_ref[...] = hc_ref[...] + hn


def _gin_final_body(agg_ref, h_ref, hc_ref, w_ref, b_ref, wo_ref, bo_ref,
                    out_ref):
    a = agg_ref[0] + agg_ref[1] + h_ref[...]
    hn = jnp.dot(a, w_ref[...], preferred_element_type=jnp.float32)
    hn = jnp.maximum(hn + b_ref[...], 0.0)
    hc = hc_ref[...] + hn
    out_ref[...] = jnp.dot(hc, wo_ref[...], preferred_element_type=jnp.float32) + bo_ref[...]


_row_spec = pl.BlockSpec((_BN, _D), lambda i: (i, 0))
_w_spec = pl.BlockSpec((_D, _D), lambda i: (0, 0))
_b_spec = pl.BlockSpec((1, _D), lambda i: (0, 0))
_agg_spec = pl.BlockSpec((_NC, _BN, _D), lambda i: (0, i, 0))
_row_shape = jax.ShapeDtypeStruct((_NP, _D), jnp.float32)


def _mlp2(x_p, W1, b1, W2, b2):
    return pl.pallas_call(
        _mlp2_body,
        grid=(_GRID,),
        in_specs=[_row_spec, _w_spec, _b_spec, _w_spec, _b_spec],
        out_specs=_row_spec,
        out_shape=_row_shape,
    )(x_p, W1, b1, W2, b2)


def _gin_layer(agg, h_p, hc_p, Wl, bl):
    return pl.pallas_call(
        _gin_body,
        grid=(_GRID,),
        in_specs=[_agg_spec, _row_spec, _row_spec, _w_spec, _b_spec],
        out_specs=[_row_spec, _row_spec],
        out_shape=[_row_shape, _row_shape],
    )(agg, h_p, hc_p, Wl, bl)


def _gin_final(agg, h_p, hc_p, Wl, bl, Wo_p, bo_p):
    return pl.pallas_call(
        _gin_final_body,
        grid=(_GRID,),
        in_specs=[_agg_spec, _row_spec, _row_spec, _w_spec, _b_spec,
                  _w_spec, _b_spec],
        out_specs=_row_spec,
        out_shape=_row_shape,
    )(agg, h_p, hc_p, Wl, bl, Wo_p, bo_p)


def kernel(x, edge_index, W1, b1, W2, b2, Wl0, bl0, Wl1, bl1, Wl2, bl2,
           Wout, bout):
    x_p = jnp.pad(x, ((0, _NP - _N), (0, 0)))
    pad_e = _EPAD - _E
    src2d = jnp.concatenate(
        [edge_index[0], jnp.full((pad_e,), _N, jnp.int32)]).reshape(
            _TOTCH, _CH)
    dst2d = jnp.concatenate(
        [edge_index[1], jnp.full((pad_e,), _N, jnp.int32)]).reshape(
            _TOTCH, _CH)
    zeros_blk = jnp.zeros((_CH, _D), jnp.float32)
    b1r = b1.reshape(1, _D)
    b2r = b2.reshape(1, _D)
    blr = [bl0.reshape(1, _D), bl1.reshape(1, _D), bl2.reshape(1, _D)]
    Wls = [Wl0, Wl1, Wl2]
    Wo_p = jnp.pad(Wout, ((0, 0), (0, _D - _T)))
    bo_p = jnp.pad(bout, (0, _D - _T)).reshape(1, _D)

    h = _mlp2(x_p, W1, b1r, W2, b2r)
    hc = h
    for layer in range(2):
        agg = _sc_aggregate(h, src2d, dst2d, zeros_blk)
        h, hc = _gin_layer(agg, h, hc, Wls[layer], blr[layer])
    agg = _sc_aggregate(h, src2d, dst2d, zeros_blk)
    out_p = _gin_final(agg, h, hc, Wls[2], blr[2], Wo_p, bo_p)
    return out_p[:_N, :_T]
